# baseline (speedup 1.0000x reference)
import jax
import jax.numpy as jnp
from jax.experimental import pallas as pl
from jax.experimental.pallas import tpu as pltpu

N_TOKENS = 32768
INPUT_DIM = 768
QUERY_DIM = 128
K_EXPERTS = 64
TOP_K = 2

TILE = 4096
CHUNK = 1024


def _gate_kernel(x_ref, w_ref, b_ref, keys_ref, probs_ref, idx_ref):
    for c in range(TILE // CHUNK):
        rows = pl.ds(c * CHUNK, CHUNK)
        query = jax.lax.dot_general(
            x_ref[rows, :], w_ref[...], (((1,), (0,)), ((), ())),
            preferred_element_type=jnp.float32,
        ) + b_ref[...]
        logits = jax.lax.dot_general(
            query, keys_ref[...], (((1,), (1,)), ((), ())),
            preferred_element_type=jnp.float32,
        ) / jnp.sqrt(jnp.float32(QUERY_DIM))
        colf = jax.lax.broadcasted_iota(jnp.int32, logits.shape, 1).astype(jnp.float32)
        big = jnp.float32(K_EXPERTS)
        l1 = jnp.max(logits, axis=1, keepdims=True)
        i1 = jnp.min(jnp.where(logits == l1, colf, big), axis=1, keepdims=True)
        masked = jnp.where(colf == i1, -jnp.inf, logits)
        l2 = jnp.max(masked, axis=1, keepdims=True)
        i2 = jnp.min(jnp.where(masked == l2, colf, big), axis=1, keepdims=True)
        e2 = jnp.exp(l2 - l1)
        denom = jnp.float32(1.0) + e2
        p1 = jnp.float32(1.0) / denom
        p2 = e2 / denom
        zero = jnp.float32(0.0)
        probs_ref[rows, :] = (jnp.where(colf == i1, p1, zero)
                              + jnp.where(colf == i2, p2, zero))
        idx_ref[rows, :] = jnp.concatenate([i1, i2], axis=1).astype(jnp.int32)

@jax.jit
def kernel(x, W, b, keys):
    b2 = b.reshape(1, QUERY_DIM)
    n_tiles = N_TOKENS // TILE
    probs, idx = pl.pallas_call(
        _gate_kernel,
        grid=(n_tiles,),
        in_specs=[
            pl.BlockSpec((TILE, INPUT_DIM), lambda i: (i, 0)),
            pl.BlockSpec((INPUT_DIM, QUERY_DIM), lambda i: (0, 0)),
            pl.BlockSpec((1, QUERY_DIM), lambda i: (0, 0)),
            pl.BlockSpec((K_EXPERTS, QUERY_DIM), lambda i: (0, 0)),
        ],
        out_specs=(
            pl.BlockSpec((TILE, K_EXPERTS), lambda i: (i, 0)),
            pl.BlockSpec((TILE, TOP_K), lambda i: (i, 0)),
        ),
        out_shape=(
            jax.ShapeDtypeStruct((N_TOKENS, K_EXPERTS), jnp.float32),
            jax.ShapeDtypeStruct((N_TOKENS, TOP_K), jnp.int32),
        ),
        compiler_params=pltpu.CompilerParams(
            dimension_semantics=("parallel",),
        ),
    )(x, W, b2, keys)
    return (probs, idx)


# speedup vs baseline: 1.0279x; 1.0279x over previous
import jax
import jax.numpy as jnp
from jax.experimental import pallas as pl
from jax.experimental.pallas import tpu as pltpu

N_TOKENS = 32768
INPUT_DIM = 768
QUERY_DIM = 128
K_EXPERTS = 64
TOP_K = 2

TILE = 4096


def _gate_kernel(x_ref, w_ref, b_ref, keys_ref, probs_ref, idx_ref):
    query = jax.lax.dot_general(
        x_ref[...], w_ref[...], (((1,), (0,)), ((), ())),
        preferred_element_type=jnp.float32,
    ) + b_ref[...]
    logits = jax.lax.dot_general(
        query, keys_ref[...], (((1,), (1,)), ((), ())),
        preferred_element_type=jnp.float32,
    ) / jnp.sqrt(jnp.float32(QUERY_DIM))
    one = jnp.float32(1.0)
    zero = jnp.float32(0.0)
    l1 = jnp.max(logits, axis=1, keepdims=True)
    oh1 = jnp.where(logits == l1, one, zero)
    masked = jnp.where(logits == l1, -jnp.inf, logits)
    l2 = jnp.max(masked, axis=1, keepdims=True)
    oh2 = jnp.where(masked == l2, one, zero)
    e2 = jnp.exp(l2 - l1)
    denom = one + e2
    p1 = one / denom
    p2 = e2 / denom
    probs_ref[...] = oh1 * p1 + oh2 * p2
    colv = jax.lax.broadcasted_iota(
        jnp.int32, (K_EXPERTS, 8), 0).astype(jnp.float32)
    comb = jax.lax.dot_general(
        oh1 + jnp.float32(K_EXPERTS) * oh2, colv, (((1,), (0,)), ((), ())),
        preferred_element_type=jnp.float32,
    )[:, 0:1]
    i2f = jnp.floor(comb * jnp.float32(1.0 / K_EXPERTS))
    i1f = comb - jnp.float32(K_EXPERTS) * i2f
    idx_ref[...] = jnp.concatenate([i1f, i2f], axis=1).astype(jnp.int32)


@jax.jit
def kernel(x, W, b, keys):
    b2 = b.reshape(1, QUERY_DIM)
    n_tiles = N_TOKENS // TILE
    probs, idx = pl.pallas_call(
        _gate_kernel,
        grid=(n_tiles,),
        in_specs=[
            pl.BlockSpec((TILE, INPUT_DIM), lambda i: (i, 0)),
            pl.BlockSpec((INPUT_DIM, QUERY_DIM), lambda i: (0, 0)),
            pl.BlockSpec((1, QUERY_DIM), lambda i: (0, 0)),
            pl.BlockSpec((K_EXPERTS, QUERY_DIM), lambda i: (0, 0)),
        ],
        out_specs=(
            pl.BlockSpec((TILE, K_EXPERTS), lambda i: (i, 0)),
            pl.BlockSpec((TILE, TOP_K), lambda i: (i, 0)),
        ),
        out_shape=(
            jax.ShapeDtypeStruct((N_TOKENS, K_EXPERTS), jnp.float32),
            jax.ShapeDtypeStruct((N_TOKENS, TOP_K), jnp.int32),
        ),
        compiler_params=pltpu.CompilerParams(
            dimension_semantics=("parallel",),
        ),
    )(x, W, b2, keys)
    return (probs, idx)
